# baseline (speedup 1.0000x reference)
import jax
import jax.numpy as jnp
from jax.experimental import pallas as pl
from jax.experimental.pallas import tpu as pltpu

Q = 1024
D = 128
K = 100000
BK = 4000
NKB = K // BK


def _knn_body(am2_ref, p2_ref, p2s_ref, inds_ref, dis_ref,
              p1sq_ref, mina_ref, idxa_ref, minb_ref, idxb_ref):
    k = pl.program_id(0)

    @pl.when(k == 0)
    def _init():
        am2 = am2_ref[...]
        p1sq_ref[...] = 0.25 * jnp.sum(am2 * am2, axis=1, keepdims=True)
        mina_ref[...] = jnp.full((Q, 1), jnp.inf, jnp.float32)
        minb_ref[...] = jnp.full((Q, 1), jnp.inf, jnp.float32)
        idxa_ref[...] = jnp.zeros((Q, 1), jnp.float32)
        idxb_ref[...] = jnp.zeros((Q, 1), jnp.float32)

    am2 = am2_ref[...]
    p1sq = p1sq_ref[...]
    base = (k * BK).astype(jnp.float32)

    def block_minarg(b):
        dot2 = jax.lax.dot_general(
            am2, b, (((1,), (1,)), ((), ())),
            preferred_element_type=jnp.float32)
        p2sq = jnp.sum(b * b, axis=1)[None, :]
        d2 = (p1sq + dot2) + p2sq
        bmin = jnp.min(d2, axis=1, keepdims=True)
        lidx = jnp.argmin(d2, axis=1).astype(jnp.float32)[:, None]
        return bmin, base + lidx

    bmin_a, bidx_a = block_minarg(p2_ref[...])
    bmin_b, bidx_b = block_minarg(p2s_ref[...])

    upd_a = bmin_a < mina_ref[...]
    mina_ref[...] = jnp.where(upd_a, bmin_a, mina_ref[...])
    idxa_ref[...] = jnp.where(upd_a, bidx_a, idxa_ref[...])
    upd_b = bmin_b < minb_ref[...]
    minb_ref[...] = jnp.where(upd_b, bmin_b, minb_ref[...])
    idxb_ref[...] = jnp.where(upd_b, bidx_b, idxb_ref[...])

    @pl.when(k == NKB - 1)
    def _fin():
        sym_mask = mina_ref[...] < minb_ref[...]
        inds_ref[...] = jnp.where(sym_mask, idxa_ref[...],
                                  idxb_ref[...]).astype(jnp.int32)
        dis_ref[...] = jnp.where(sym_mask, mina_ref[...], minb_ref[...])


@jax.jit
def _run(am2, p2, p2s):
    return pl.pallas_call(
        _knn_body,
        grid=(NKB,),
        in_specs=[
            pl.BlockSpec((Q, D), lambda k: (0, 0)),
            pl.BlockSpec((BK, D), lambda k: (k, 0)),
            pl.BlockSpec((BK, D), lambda k: (k, 0)),
        ],
        out_specs=[
            pl.BlockSpec((Q, 1), lambda k: (0, 0)),
            pl.BlockSpec((Q, 1), lambda k: (0, 0)),
        ],
        out_shape=[
            jax.ShapeDtypeStruct((Q, 1), jnp.int32),
            jax.ShapeDtypeStruct((Q, 1), jnp.float32),
        ],
        scratch_shapes=[
            pltpu.VMEM((Q, 1), jnp.float32),
            pltpu.VMEM((Q, 1), jnp.float32),
            pltpu.VMEM((Q, 1), jnp.float32),
            pltpu.VMEM((Q, 1), jnp.float32),
            pltpu.VMEM((Q, 1), jnp.float32),
        ],
    )(am2, p2, p2s)


def kernel(p1_key_points, p2_key_points, p2_key_points_sym):
    am2 = -2.0 * p1_key_points[0]
    inds, dis = _run(am2, p2_key_points[0], p2_key_points_sym[0])
    return inds[None].astype(jnp.int64), dis[None]


# speedup vs baseline: 1.1802x; 1.1802x over previous
import jax
import jax.numpy as jnp
from jax.experimental import pallas as pl
from jax.experimental.pallas import tpu as pltpu

Q = 1024
D = 128
K = 100000
BK = 5000
NKB = K // BK


def _knn_body(am2_ref, p2_ref, p2s_ref, inds_ref, dis_ref,
              p1sq_ref, mina_ref, idxa_ref, minb_ref, idxb_ref):
    k = pl.program_id(0)

    @pl.when(k == 0)
    def _init():
        am2 = am2_ref[...]
        p1sq_ref[...] = 0.25 * jnp.sum(am2 * am2, axis=1, keepdims=True)
        mina_ref[...] = jnp.full((Q, 1), jnp.inf, jnp.float32)
        minb_ref[...] = jnp.full((Q, 1), jnp.inf, jnp.float32)
        idxa_ref[...] = jnp.zeros((Q, 1), jnp.float32)
        idxb_ref[...] = jnp.zeros((Q, 1), jnp.float32)

    am2 = am2_ref[...]
    p1sq = p1sq_ref[...]
    iota_f = jax.lax.broadcasted_iota(jnp.int32, (Q, BK), 1).astype(jnp.float32)
    base = (k * BK).astype(jnp.float32)

    def block_minarg(b):
        dot2 = jax.lax.dot_general(
            am2, b, (((1,), (1,)), ((), ())),
            preferred_element_type=jnp.float32)
        p2sq = jnp.sum(b * b, axis=1)[None, :]
        d2 = (p1sq + dot2) + p2sq
        bmin = jnp.min(d2, axis=1, keepdims=True)
        lidx = jnp.min(jnp.where(d2 == bmin, iota_f, jnp.float32(3e38)),
                       axis=1, keepdims=True)
        return bmin, base + lidx

    bmin_a, bidx_a = block_minarg(p2_ref[...])
    bmin_b, bidx_b = block_minarg(p2s_ref[...])

    upd_a = bmin_a < mina_ref[...]
    mina_ref[...] = jnp.where(upd_a, bmin_a, mina_ref[...])
    idxa_ref[...] = jnp.where(upd_a, bidx_a, idxa_ref[...])
    upd_b = bmin_b < minb_ref[...]
    minb_ref[...] = jnp.where(upd_b, bmin_b, minb_ref[...])
    idxb_ref[...] = jnp.where(upd_b, bidx_b, idxb_ref[...])

    @pl.when(k == NKB - 1)
    def _fin():
        sym_mask = mina_ref[...] < minb_ref[...]
        inds_ref[...] = jnp.where(sym_mask, idxa_ref[...],
                                  idxb_ref[...]).astype(jnp.int32)
        dis_ref[...] = jnp.where(sym_mask, mina_ref[...], minb_ref[...])


@jax.jit
def _run(am2, p2, p2s):
    return pl.pallas_call(
        _knn_body,
        grid=(NKB,),
        in_specs=[
            pl.BlockSpec((Q, D), lambda k: (0, 0)),
            pl.BlockSpec((BK, D), lambda k: (k, 0)),
            pl.BlockSpec((BK, D), lambda k: (k, 0)),
        ],
        out_specs=[
            pl.BlockSpec((Q, 1), lambda k: (0, 0)),
            pl.BlockSpec((Q, 1), lambda k: (0, 0)),
        ],
        out_shape=[
            jax.ShapeDtypeStruct((Q, 1), jnp.int32),
            jax.ShapeDtypeStruct((Q, 1), jnp.float32),
        ],
        scratch_shapes=[
            pltpu.VMEM((Q, 1), jnp.float32),
            pltpu.VMEM((Q, 1), jnp.float32),
            pltpu.VMEM((Q, 1), jnp.float32),
            pltpu.VMEM((Q, 1), jnp.float32),
            pltpu.VMEM((Q, 1), jnp.float32),
        ],
    )(am2, p2, p2s)


def kernel(p1_key_points, p2_key_points, p2_key_points_sym):
    am2 = -2.0 * p1_key_points[0]
    inds, dis = _run(am2, p2_key_points[0], p2_key_points_sym[0])
    return inds[None].astype(jnp.int64), dis[None]
